# Initial kernel scaffold; baseline (speedup 1.0000x reference)
import jax
import jax.numpy as jnp
from jax.experimental import pallas as pl


def kernel(x, edge_index, batch, Wl1, bl1, Wr1, Wl2, bl2, Wr2, Wl3, bl3, Wr3, Wc, bc):
    raise NotImplementedError("write your pallas kernel here")


# speedup vs baseline: 7.4800x; 7.4800x over previous
import functools

import jax
import jax.numpy as jnp
from jax import lax
from jax.experimental import pallas as pl
from jax.experimental.pallas import tpu as pltpu
from jax.experimental.pallas import tpu_sc as plsc

_N = 10000
_E = 320000
_NPAD = 10240
_NW = 32
_C = 128
_NCHUNK = 80
_EPAD = _C * _NCHUNK * _NW
_RPT = _NPAD // 16
_D = 128
_MB = 512
_GRID_M = _NPAD // _MB

_mesh = plsc.VectorSubcoreMesh(core_axis_name="c", subcore_axis_name="s")


@functools.partial(
    pl.kernel, mesh=_mesh,
    out_type=jax.ShapeDtypeStruct((2, _NPAD, _D), jnp.float32),
    scratch_types=[
        pltpu.VMEM((_NCHUNK, _C), jnp.int32),
        pltpu.VMEM((_NCHUNK, _C), jnp.int32),
        pltpu.VMEM((_C, _D), jnp.float32),
        pltpu.VMEM_SHARED((_NPAD, _D), jnp.float32),
    ])
def _agg(vals_hbm, src_hbm, dst_hbm, part_hbm, src_v, dst_v, rows_v, acc_sh):
  cid = lax.axis_index("c")
  sid = lax.axis_index("s")
  wid = cid * 16 + sid
  nrc = _RPT // _C
  zero16 = jnp.zeros((16,), jnp.float32)

  pltpu.sync_copy(src_hbm.at[wid], src_v)
  pltpu.sync_copy(dst_hbm.at[wid], dst_v)

  @pl.loop(0, _C)
  def _(i):
    @pl.loop(0, _D, step=16)
    def _(j):
      rows_v[i, pl.ds(j, 16)] = zero16
  for t in range(nrc):
    pltpu.sync_copy(rows_v, acc_sh.at[pl.ds(sid * _RPT + t * _C, _C)])

  plsc.subcore_barrier()

  @pl.loop(0, _NCHUNK)
  def _(j):
    pltpu.sync_copy(vals_hbm.at[src_v.at[j]], rows_v)
    pltpu.sync_copy(rows_v, acc_sh.at[dst_v.at[j]], add=True)

  plsc.subcore_barrier()

  for t in range(nrc):
    rs = pl.ds(sid * _RPT + t * _C, _C)
    pltpu.sync_copy(acc_sh.at[rs], rows_v)
    pltpu.sync_copy(rows_v, part_hbm.at[cid, rs])


def _tc_proj1(xp, Wl1, Wr1, bl1r):
  def body(x_ref, wl_ref, wr_ref, b_ref, p_ref, r_ref):
    xb = x_ref[...]
    p1 = jnp.dot(xb, wl_ref[...], preferred_element_type=jnp.float32)
    ones = jnp.ones((_MB, 1), jnp.float32)
    zeros = jnp.zeros((_MB, _D - 17), jnp.float32)
    p_ref[...] = jnp.concatenate([p1, ones, zeros], axis=1)
    r_ref[...] = (jnp.dot(xb, wr_ref[...], preferred_element_type=jnp.float32)
                  + b_ref[...])
  return pl.pallas_call(
      body,
      grid=(_GRID_M,),
      in_specs=[pl.BlockSpec((_MB, 128), lambda i: (i, 0)),
                pl.BlockSpec((128, 16), lambda i: (0, 0)),
                pl.BlockSpec((128, 16), lambda i: (0, 0)),
                pl.BlockSpec((1, 16), lambda i: (0, 0))],
      out_specs=[pl.BlockSpec((_MB, _D), lambda i: (i, 0)),
                 pl.BlockSpec((_MB, 16), lambda i: (i, 0))],
      out_shape=(jax.ShapeDtypeStruct((_NPAD, _D), jnp.float32),
                 jax.ShapeDtypeStruct((_NPAD, 16), jnp.float32)),
  )(xp, Wl1, Wr1, bl1r)


def _tc_layer1(part1, r1):
  def body(p_ref, r_ref, h_ref, c_ref):
    s = p_ref[0] + p_ref[1]
    cnt = jnp.maximum(s[:, 16:17], 1.0)
    mean = s[:, 0:16] / cnt
    h = jnp.maximum(mean + r_ref[...], 0.0)
    h_ref[...] = jnp.concatenate(
        [h, jnp.zeros((_MB, _D - 16), jnp.float32)], axis=1)
    c_ref[...] = jnp.broadcast_to(cnt, (_MB, 16))
  return pl.pallas_call(
      body,
      grid=(_GRID_M,),
      in_specs=[pl.BlockSpec((2, _MB, _D), lambda i: (0, i, 0)),
                pl.BlockSpec((_MB, 16), lambda i: (i, 0))],
      out_specs=[pl.BlockSpec((_MB, _D), lambda i: (i, 0)),
                 pl.BlockSpec((_MB, 16), lambda i: (i, 0))],
      out_shape=(jax.ShapeDtypeStruct((_NPAD, _D), jnp.float32),
                 jax.ShapeDtypeStruct((_NPAD, 16), jnp.float32)),
  )(part1, r1)


def _tc_layer2(part2, cnt16, h128, Wl2, Wr2, bl2r):
  def body(p_ref, c_ref, h_ref, wl_ref, wr_ref, b_ref, color_ref, ps_ref):
    i = pl.program_id(0)
    cnt = c_ref[:, 0:1]
    mean = (p_ref[0] + p_ref[1])[:, 0:16] / cnt
    h2 = (jnp.dot(mean, wl_ref[...], preferred_element_type=jnp.float32)
          + jnp.dot(h_ref[:, 0:16], wr_ref[...],
                    preferred_element_type=jnp.float32)
          + b_ref[...])
    color = jnp.maximum(h2, 0.0)
    color_ref[...] = jnp.concatenate(
        [color, jnp.zeros((_MB, _D - 32), jnp.float32)], axis=1)
    row = i * _MB + lax.broadcasted_iota(jnp.int32, (_MB, 1), 0)
    ps_ref[0] = jnp.sum(jnp.where(row < _N, h2, 0.0), axis=0, keepdims=True)
  return pl.pallas_call(
      body,
      grid=(_GRID_M,),
      in_specs=[pl.BlockSpec((2, _MB, _D), lambda i: (0, i, 0)),
                pl.BlockSpec((_MB, 16), lambda i: (i, 0)),
                pl.BlockSpec((_MB, _D), lambda i: (i, 0)),
                pl.BlockSpec((16, 32), lambda i: (0, 0)),
                pl.BlockSpec((16, 32), lambda i: (0, 0)),
                pl.BlockSpec((1, 32), lambda i: (0, 0))],
      out_specs=[pl.BlockSpec((_MB, _D), lambda i: (i, 0)),
                 pl.BlockSpec((1, 1, 32), lambda i: (i, 0, 0))],
      out_shape=(jax.ShapeDtypeStruct((_NPAD, _D), jnp.float32),
                 jax.ShapeDtypeStruct((_GRID_M, 1, 32), jnp.float32)),
  )(part2, cnt16, h128, Wl2, Wr2, bl2r)


def _tc_layer3(part3, cnt16, color128, psum, Wl3p, Wr3p, bl3r, Wcp, bcr):
  def body(p_ref, c_ref, col_ref, ps_ref, wl_ref, wr_ref, b_ref, wc_ref,
           bc_ref, out_ref, cls_ref):
    i = pl.program_id(0)
    cnt = c_ref[:, 0:1]
    mean = (p_ref[0] + p_ref[1])[:, 0:32] / cnt
    out_ref[...] = (
        jnp.dot(mean, wl_ref[...], preferred_element_type=jnp.float32)
        + jnp.dot(col_ref[:, 0:32], wr_ref[...],
                  preferred_element_type=jnp.float32)
        + b_ref[...])
    @pl.when(i == 0)
    def _():
      pooled = jnp.sum(ps_ref[...], axis=(0, 1)).reshape(1, 32) * (1.0 / _N)
      cls_ref[...] = (jnp.dot(pooled, wc_ref[...],
                              preferred_element_type=jnp.float32)
                      + bc_ref[...])
  return pl.pallas_call(
      body,
      grid=(_GRID_M,),
      in_specs=[pl.BlockSpec((2, _MB, _D), lambda i: (0, i, 0)),
                pl.BlockSpec((_MB, 16), lambda i: (i, 0)),
                pl.BlockSpec((_MB, _D), lambda i: (i, 0)),
                pl.BlockSpec((_GRID_M, 1, 32), lambda i: (0, 0, 0)),
                pl.BlockSpec((32, 32), lambda i: (0, 0)),
                pl.BlockSpec((32, 32), lambda i: (0, 0)),
                pl.BlockSpec((1, 32), lambda i: (0, 0)),
                pl.BlockSpec((32, 16), lambda i: (0, 0)),
                pl.BlockSpec((1, 16), lambda i: (0, 0))],
      out_specs=[pl.BlockSpec((_MB, 32), lambda i: (i, 0)),
                 pl.BlockSpec((1, 16), lambda i: (0, 0))],
      out_shape=(jax.ShapeDtypeStruct((_NPAD, 32), jnp.float32),
                 jax.ShapeDtypeStruct((1, 16), jnp.float32)),
  )(part3, cnt16, color128, psum, Wl3p, Wr3p, bl3r, Wcp, bcr)


def kernel(x, edge_index, batch, Wl1, bl1, Wr1, Wl2, bl2, Wr2, Wl3, bl3, Wr3,
           Wc, bc):
  xp = jnp.pad(x, ((0, _NPAD - _N), (0, 0)))
  src = edge_index[0]
  dst = edge_index[1]
  npad = _EPAD - _E
  fill = jnp.arange(npad, dtype=jnp.int32)
  src3 = jnp.concatenate([src, fill % _N]).reshape(_NW, _NCHUNK, _C)
  dst3 = jnp.concatenate([dst, _N + fill % (_NPAD - _N)]).reshape(
      _NW, _NCHUNK, _C)

  p1aug, r1 = _tc_proj1(xp, Wl1, Wr1, bl1.reshape(1, 16))
  part1 = _agg(p1aug, src3, dst3)
  h128, cnt16 = _tc_layer1(part1, r1)
  part2 = _agg(h128, src3, dst3)
  color128, psum = _tc_layer2(part2, cnt16, h128, Wl2, Wr2, bl2.reshape(1, 32))
  part3 = _agg(color128, src3, dst3)
  Wl3p = jnp.pad(Wl3, ((0, 0), (0, 11)))
  Wr3p = jnp.pad(Wr3, ((0, 0), (0, 11)))
  bl3r = jnp.pad(bl3, (0, 11)).reshape(1, 32)
  Wcp = jnp.pad(Wc, ((0, 0), (0, 6)))
  bcr = jnp.pad(bc, (0, 6)).reshape(1, 16)
  color_out, cls = _tc_layer3(part3, cnt16, color128, psum, Wl3p, Wr3p, bl3r,
                              Wcp, bcr)
  return (cls[:, :10], color_out[:_N, :21])


# speedup vs baseline: 9.6638x; 1.2919x over previous
import functools

import jax
import jax.numpy as jnp
from jax import lax
from jax.experimental import pallas as pl
from jax.experimental.pallas import tpu as pltpu
from jax.experimental.pallas import tpu_sc as plsc

_N = 10000
_E = 320000
_NPAD = 10240
_NW = 32
_C = 128
_NCHUNK = 80
_EPAD = _C * _NCHUNK * _NW
_RPT = _NPAD // 16
_D = 128
_MB = 512
_GRID_M = _NPAD // _MB

_mesh = plsc.VectorSubcoreMesh(core_axis_name="c", subcore_axis_name="s")


@functools.partial(
    pl.kernel, mesh=_mesh,
    out_type=jax.ShapeDtypeStruct((2, _NPAD, _D), jnp.float32),
    scratch_types=[
        pltpu.VMEM((_NCHUNK, _C), jnp.int32),
        pltpu.VMEM((8, _C), jnp.int32),
        pltpu.VMEM((8, _C), jnp.int32),
        pltpu.VMEM((_C, _D), jnp.float32),
        pltpu.VMEM((_C, _D), jnp.float32),
        pltpu.VMEM_SHARED((_NPAD, _D), jnp.float32),
        pltpu.SemaphoreType.DMA,
        pltpu.SemaphoreType.DMA,
        pltpu.SemaphoreType.DMA,
        pltpu.SemaphoreType.DMA,
        pltpu.SemaphoreType.DMA,
        pltpu.SemaphoreType.DMA,
    ])
def _agg(vals_hbm, src_hbm, dst_hbm, part_hbm, dst_v, si0, si1, r0, r1,
         acc_sh, ssi0, ssi1, sg0, sg1, ss0, ss1):
  cid = lax.axis_index("c")
  sid = lax.axis_index("s")
  wid = cid * 16 + sid
  nrc = _RPT // _C
  zero16 = jnp.zeros((16,), jnp.float32)
  srcs = (si0, si1)
  rows = (r0, r1)
  sis = (ssi0, ssi1)
  sgs = (sg0, sg1)
  sss = (ss0, ss1)

  def src_copy(j, b):
    return pltpu.make_async_copy(src_hbm.at[wid, pl.ds(j, 1)],
                                 srcs[b].at[pl.ds(0, 1)], sis[b])

  def gather(j_ref_b, b):
    return pltpu.make_async_copy(vals_hbm.at[srcs[b].at[0]], rows[b], sgs[b])

  def scatter(j, b):
    return pltpu.make_async_copy(rows[b], acc_sh.at[dst_v.at[j]], sss[b])

  pltpu.sync_copy(dst_hbm.at[wid], dst_v)
  src_copy(0, 0).start()
  src_copy(1, 1).start()

  @pl.loop(0, _C)
  def _(i):
    @pl.loop(0, _D, step=16)
    def _(j):
      r0[i, pl.ds(j, 16)] = zero16
  for t in range(nrc):
    pltpu.sync_copy(r0, acc_sh.at[pl.ds(sid * _RPT + t * _C, _C)])

  plsc.subcore_barrier()

  @pl.loop(0, _NCHUNK // 2)
  def _(u):
    for b in range(2):
      j = 2 * u + b
      src_copy(j, b).wait()
      @pl.when(u != 0)
      def _():
        scatter(j, b).wait()
      gather(j, b).start()
      gather(j, b).wait()
      @pl.when(j + 2 < _NCHUNK)
      def _():
        src_copy(j + 2, b).start()
      scatter(j, b).start()

  scatter(_NCHUNK - 2, 0).wait()
  scatter(_NCHUNK - 1, 1).wait()

  plsc.subcore_barrier()

  for t in range(nrc):
    rs = pl.ds(sid * _RPT + t * _C, _C)
    pltpu.sync_copy(acc_sh.at[rs], r0)
    pltpu.sync_copy(r0, part_hbm.at[cid, rs])


def _tc_proj1(xp, Wl1, Wr1, bl1r):
  def body(x_ref, wl_ref, wr_ref, b_ref, p_ref, r_ref):
    xb = x_ref[...]
    p1 = jnp.dot(xb, wl_ref[...], preferred_element_type=jnp.float32)
    ones = jnp.ones((_MB, 1), jnp.float32)
    zeros = jnp.zeros((_MB, _D - 17), jnp.float32)
    p_ref[...] = jnp.concatenate([p1, ones, zeros], axis=1)
    r_ref[...] = (jnp.dot(xb, wr_ref[...], preferred_element_type=jnp.float32)
                  + b_ref[...])
  return pl.pallas_call(
      body,
      grid=(_GRID_M,),
      in_specs=[pl.BlockSpec((_MB, 128), lambda i: (i, 0)),
                pl.BlockSpec((128, 16), lambda i: (0, 0)),
                pl.BlockSpec((128, 16), lambda i: (0, 0)),
                pl.BlockSpec((1, 16), lambda i: (0, 0))],
      out_specs=[pl.BlockSpec((_MB, _D), lambda i: (i, 0)),
                 pl.BlockSpec((_MB, 16), lambda i: (i, 0))],
      out_shape=(jax.ShapeDtypeStruct((_NPAD, _D), jnp.float32),
                 jax.ShapeDtypeStruct((_NPAD, 16), jnp.float32)),
  )(xp, Wl1, Wr1, bl1r)


def _tc_layer1(part1, r1):
  def body(p_ref, r_ref, h_ref, c_ref):
    s = p_ref[0] + p_ref[1]
    cnt = jnp.maximum(s[:, 16:17], 1.0)
    mean = s[:, 0:16] / cnt
    h = jnp.maximum(mean + r_ref[...], 0.0)
    h_ref[...] = jnp.concatenate(
        [h, jnp.zeros((_MB, _D - 16), jnp.float32)], axis=1)
    c_ref[...] = jnp.broadcast_to(cnt, (_MB, 16))
  return pl.pallas_call(
      body,
      grid=(_GRID_M,),
      in_specs=[pl.BlockSpec((2, _MB, _D), lambda i: (0, i, 0)),
                pl.BlockSpec((_MB, 16), lambda i: (i, 0))],
      out_specs=[pl.BlockSpec((_MB, _D), lambda i: (i, 0)),
                 pl.BlockSpec((_MB, 16), lambda i: (i, 0))],
      out_shape=(jax.ShapeDtypeStruct((_NPAD, _D), jnp.float32),
                 jax.ShapeDtypeStruct((_NPAD, 16), jnp.float32)),
  )(part1, r1)


def _tc_layer2(part2, cnt16, h128, Wl2, Wr2, bl2r):
  def body(p_ref, c_ref, h_ref, wl_ref, wr_ref, b_ref, color_ref, ps_ref):
    i = pl.program_id(0)
    cnt = c_ref[:, 0:1]
    mean = (p_ref[0] + p_ref[1])[:, 0:16] / cnt
    h2 = (jnp.dot(mean, wl_ref[...], preferred_element_type=jnp.float32)
          + jnp.dot(h_ref[:, 0:16], wr_ref[...],
                    preferred_element_type=jnp.float32)
          + b_ref[...])
    color = jnp.maximum(h2, 0.0)
    color_ref[...] = jnp.concatenate(
        [color, jnp.zeros((_MB, _D - 32), jnp.float32)], axis=1)
    row = i * _MB + lax.broadcasted_iota(jnp.int32, (_MB, 1), 0)
    ps_ref[0] = jnp.sum(jnp.where(row < _N, h2, 0.0), axis=0, keepdims=True)
  return pl.pallas_call(
      body,
      grid=(_GRID_M,),
      in_specs=[pl.BlockSpec((2, _MB, _D), lambda i: (0, i, 0)),
                pl.BlockSpec((_MB, 16), lambda i: (i, 0)),
                pl.BlockSpec((_MB, _D), lambda i: (i, 0)),
                pl.BlockSpec((16, 32), lambda i: (0, 0)),
                pl.BlockSpec((16, 32), lambda i: (0, 0)),
                pl.BlockSpec((1, 32), lambda i: (0, 0))],
      out_specs=[pl.BlockSpec((_MB, _D), lambda i: (i, 0)),
                 pl.BlockSpec((1, 1, 32), lambda i: (i, 0, 0))],
      out_shape=(jax.ShapeDtypeStruct((_NPAD, _D), jnp.float32),
                 jax.ShapeDtypeStruct((_GRID_M, 1, 32), jnp.float32)),
  )(part2, cnt16, h128, Wl2, Wr2, bl2r)


def _tc_layer3(part3, cnt16, color128, psum, Wl3p, Wr3p, bl3r, Wcp, bcr):
  def body(p_ref, c_ref, col_ref, ps_ref, wl_ref, wr_ref, b_ref, wc_ref,
           bc_ref, out_ref, cls_ref):
    i = pl.program_id(0)
    cnt = c_ref[:, 0:1]
    mean = (p_ref[0] + p_ref[1])[:, 0:32] / cnt
    out_ref[...] = (
        jnp.dot(mean, wl_ref[...], preferred_element_type=jnp.float32)
        + jnp.dot(col_ref[:, 0:32], wr_ref[...],
                  preferred_element_type=jnp.float32)
        + b_ref[...])
    @pl.when(i == 0)
    def _():
      pooled = jnp.sum(ps_ref[...], axis=(0, 1)).reshape(1, 32) * (1.0 / _N)
      cls_ref[...] = (jnp.dot(pooled, wc_ref[...],
                              preferred_element_type=jnp.float32)
                      + bc_ref[...])
  return pl.pallas_call(
      body,
      grid=(_GRID_M,),
      in_specs=[pl.BlockSpec((2, _MB, _D), lambda i: (0, i, 0)),
                pl.BlockSpec((_MB, 16), lambda i: (i, 0)),
                pl.BlockSpec((_MB, _D), lambda i: (i, 0)),
                pl.BlockSpec((_GRID_M, 1, 32), lambda i: (0, 0, 0)),
                pl.BlockSpec((32, 32), lambda i: (0, 0)),
                pl.BlockSpec((32, 32), lambda i: (0, 0)),
                pl.BlockSpec((1, 32), lambda i: (0, 0)),
                pl.BlockSpec((32, 16), lambda i: (0, 0)),
                pl.BlockSpec((1, 16), lambda i: (0, 0))],
      out_specs=[pl.BlockSpec((_MB, 32), lambda i: (i, 0)),
                 pl.BlockSpec((1, 16), lambda i: (0, 0))],
      out_shape=(jax.ShapeDtypeStruct((_NPAD, 32), jnp.float32),
                 jax.ShapeDtypeStruct((1, 16), jnp.float32)),
  )(part3, cnt16, color128, psum, Wl3p, Wr3p, bl3r, Wcp, bcr)


def kernel(x, edge_index, batch, Wl1, bl1, Wr1, Wl2, bl2, Wr2, Wl3, bl3, Wr3,
           Wc, bc):
  xp = jnp.pad(x, ((0, _NPAD - _N), (0, 0)))
  src = edge_index[0]
  dst = edge_index[1]
  npad = _EPAD - _E
  fill = jnp.arange(npad, dtype=jnp.int32)
  src3 = jnp.concatenate([src, fill % _N]).reshape(_NW, _NCHUNK, _C)
  dst3 = jnp.concatenate([dst, _N + fill % (_NPAD - _N)]).reshape(
      _NW, _NCHUNK, _C)

  p1aug, r1 = _tc_proj1(xp, Wl1, Wr1, bl1.reshape(1, 16))
  part1 = _agg(p1aug, src3, dst3)
  h128, cnt16 = _tc_layer1(part1, r1)
  part2 = _agg(h128, src3, dst3)
  color128, psum = _tc_layer2(part2, cnt16, h128, Wl2, Wr2, bl2.reshape(1, 32))
  part3 = _agg(color128, src3, dst3)
  Wl3p = jnp.pad(Wl3, ((0, 0), (0, 11)))
  Wr3p = jnp.pad(Wr3, ((0, 0), (0, 11)))
  bl3r = jnp.pad(bl3, (0, 11)).reshape(1, 32)
  Wcp = jnp.pad(Wc, ((0, 0), (0, 6)))
  bcr = jnp.pad(bc, (0, 6)).reshape(1, 16)
  color_out, cls = _tc_layer3(part3, cnt16, color128, psum, Wl3p, Wr3p, bl3r,
                              Wcp, bcr)
  return (cls[:, :10], color_out[:_N, :21])
